# baseline (speedup 1.0000x reference)
import functools
import math

import jax
import jax.numpy as jnp
from jax.experimental import pallas as pl
from jax.experimental.pallas import tpu as pltpu

_B = 8192
_IN = 128
_HID = 128
_OUT = 128
_NN = 12
_TB = 256
_RSQ = 1.0 / math.sqrt(128.0)


def _fused_kernel(c_ref, x_ref, w1_ref, b1_ref, w2_ref, b2_ref,
                  z_ref, zw_ref, zb_ref,
                  l1ws_ref, l1wa_ref, l1b_ref,
                  l2ws_ref, l2wa_ref, l2b_ref, o_ref):
    x = x_ref[...]
    h1 = jnp.maximum(x @ w1_ref[...] + b1_ref[...], 0.0)
    hs = []
    for n in range(_NN):
        hn = h1[:, n * _HID:(n + 1) * _HID]
        hs.append(hn @ w2_ref[n * _HID:(n + 1) * _HID, :] + b2_ref[n:n + 1, :])

    zfeat = z_ref[...] @ zw_ref[...] + zb_ref[...]
    crows = [c_ref[n:n + 1, :] for n in range(_NN)]

    def gnn_layer(hcur, ws, wa, bias, use_relu):
        gp = {}
        for n in range(_NN):
            for m in range(n, _NN):
                gp[(n, m)] = jnp.sum(hcur[n] * hcur[m], axis=1, keepdims=True)

        def score(n, m):
            return gp[(n, m) if n <= m else (m, n)]

        aggs = []
        for n in range(_NN):
            sn = jnp.concatenate([score(n, m) for m in range(_NN)],
                                 axis=1) * _RSQ
            crow = crows[n]
            mx = jnp.max(jnp.where(crow > 0, sn, -jnp.inf),
                         axis=1, keepdims=True)
            mx = jnp.where(jnp.isfinite(mx), mx, 0.0)
            cex = crow * jnp.exp(sn - mx)
            den = jnp.sum(cex, axis=1, keepdims=True)
            w = cex * (1.0 / (den + 1e-9))
            acc = w[:, 0:1] * hcur[0]
            for m in range(1, _NN):
                acc = acc + w[:, m:m + 1] * hcur[m]
            aggs.append(acc)

        out = []
        for n in range(_NN):
            o = hcur[n] @ ws + aggs[n] @ wa + bias
            if use_relu:
                o = jnp.maximum(o, 0.0)
            out.append(o)
        return out

    hs = gnn_layer(hs, l1ws_ref[...], l1wa_ref[...],
                   l1b_ref[...] + zfeat, True)
    hs = gnn_layer(hs, l2ws_ref[...], l2wa_ref[...], l2b_ref[...], False)

    acc = hs[0]
    for n in range(1, _NN):
        acc = acc + hs[n]
    o_ref[...] = acc * (1.0 / _NN)


def _full(shape):
    zeros = (0,) * len(shape)
    return pl.BlockSpec(shape, lambda i, z=zeros: z)


@jax.jit
def _run(c, x, w1cat, b1cat, w2cat, b2, z2, zw, zb, l1ws, l1wa, l1b, l2ws, l2wa, l2b):
    return pl.pallas_call(
        _fused_kernel,
        grid=(_B // _TB,),
        in_specs=[
            _full((_NN, _NN)),
            pl.BlockSpec((_TB, _IN), lambda i: (i, 0)),
            _full((_IN, _NN * _HID)),
            _full((1, _NN * _HID)),
            _full((_NN * _HID, _OUT)),
            _full((_NN, _OUT)),
            _full((1, 16)),
            _full((16, _HID)),
            _full((1, _HID)),
            _full((_OUT, _HID)),
            _full((_OUT, _HID)),
            _full((1, _HID)),
            _full((_HID, _OUT)),
            _full((_HID, _OUT)),
            _full((1, _OUT)),
        ],
        out_specs=pl.BlockSpec((_TB, _OUT), lambda i: (i, 0)),
        out_shape=jax.ShapeDtypeStruct((_B, _OUT), jnp.float32),
        compiler_params=pltpu.CompilerParams(
            dimension_semantics=("parallel",)),
    )(c, x, w1cat, b1cat, w2cat, b2, z2, zw, zb, l1ws, l1wa, l1b, l2ws, l2wa, l2b)


def kernel(x, edge_index, z, params):
    src = edge_index[0]
    dst = edge_index[1]
    c = jnp.zeros((_NN, _NN), jnp.float32).at[dst, src].add(1.0)
    w1cat = params["gauge_W1"].transpose(1, 0, 2).reshape(_IN, _NN * _HID)
    b1cat = params["gauge_b1"].reshape(1, _NN * _HID)
    w2cat = params["gauge_W2"].reshape(_NN * _HID, _OUT)
    return _run(
        c, x, w1cat, b1cat, w2cat, params["gauge_b2"],
        z.reshape(1, 16), params["z_W"], params["z_b"].reshape(1, _HID),
        params["l1_Ws"], params["l1_Wa"], params["l1_b"].reshape(1, _HID),
        params["l2_Ws"], params["l2_Wa"], params["l2_b"].reshape(1, _OUT),
    )


# speedup vs baseline: 3.9629x; 3.9629x over previous
import functools
import math

import jax
import jax.numpy as jnp
from jax.experimental import pallas as pl
from jax.experimental.pallas import tpu as pltpu

_B = 8192
_IN = 128
_HID = 128
_OUT = 128
_NN = 12
_TB = 256
_RSQ = 1.0 / math.sqrt(128.0)


def _fused_kernel(c_ref, xt_ref, w1t_ref, b1t_ref, w2t_ref, b2t_ref,
                  zft_ref,
                  l1wst_ref, l1wat_ref, l1bt_ref,
                  l2wst_ref, l2wat_ref, l2bt_ref, o_ref):
    xt = xt_ref[...]
    h1 = jnp.maximum(w1t_ref[...] @ xt + b1t_ref[...], 0.0)
    hs = []
    for n in range(_NN):
        h1n = h1[n * _HID:(n + 1) * _HID, :]
        hs.append(w2t_ref[n * _HID:(n + 1) * _HID, :] @ h1n
                  + b2t_ref[:, n:n + 1])

    ones_row = jnp.ones((1, _HID), jnp.float32)
    bias1 = l1bt_ref[...] + zft_ref[...]

    def gnn_layer(hcur, wst, wat, bias, use_relu):
        gp = {}
        for n in range(_NN):
            for m in range(n, _NN):
                gp[(n, m)] = (ones_row @ (hcur[n] * hcur[m])) * _RSQ

        def score(n, m):
            return gp[(n, m) if n <= m else (m, n)]

        aggs = []
        for n in range(_NN):
            cs = [c_ref[n, m] for m in range(_NN)]
            mx = jnp.full((1, _TB), -jnp.inf, jnp.float32)
            for m in range(_NN):
                mx = jnp.where(cs[m] > 0, jnp.maximum(mx, score(n, m)), mx)
            mx = jnp.where(jnp.isfinite(mx), mx, 0.0)
            exs = []
            den = jnp.zeros((1, _TB), jnp.float32)
            for m in range(_NN):
                e = jnp.exp(score(n, m) - mx)
                exs.append(e)
                den = den + cs[m] * e
            inv = 1.0 / (den + 1e-9)
            acc = ((cs[0] * exs[0]) * inv) * hcur[0]
            for m in range(1, _NN):
                acc = acc + ((cs[m] * exs[m]) * inv) * hcur[m]
            aggs.append(acc)

        out = []
        for n in range(_NN):
            o = wst @ hcur[n] + wat @ aggs[n] + bias
            if use_relu:
                o = jnp.maximum(o, 0.0)
            out.append(o)
        return out

    hs = gnn_layer(hs, l1wst_ref[...], l1wat_ref[...], bias1, True)
    hs = gnn_layer(hs, l2wst_ref[...], l2wat_ref[...], l2bt_ref[...], False)

    acc = hs[0]
    for n in range(1, _NN):
        acc = acc + hs[n]
    o_ref[...] = acc * (1.0 / _NN)


def _full(shape):
    zeros = (0,) * len(shape)
    return pl.BlockSpec(shape, lambda i, z=zeros: z)


@jax.jit
def _run(c, xt, w1t, b1t, w2t, b2t, zft, l1wst, l1wat, l1bt, l2wst, l2wat, l2bt):
    out_t = pl.pallas_call(
        _fused_kernel,
        grid=(_B // _TB,),
        in_specs=[
            pl.BlockSpec(memory_space=pltpu.SMEM),
            pl.BlockSpec((_IN, _TB), lambda i: (0, i)),
            _full((_NN * _HID, _IN)),
            _full((_NN * _HID, 1)),
            _full((_NN * _HID, _HID)),
            _full((_OUT, _NN)),
            _full((_HID, 1)),
            _full((_HID, _OUT)),
            _full((_HID, _OUT)),
            _full((_HID, 1)),
            _full((_OUT, _HID)),
            _full((_OUT, _HID)),
            _full((_OUT, 1)),
        ],
        out_specs=pl.BlockSpec((_OUT, _TB), lambda i: (0, i)),
        out_shape=jax.ShapeDtypeStruct((_OUT, _B), jnp.float32),
        compiler_params=pltpu.CompilerParams(
            dimension_semantics=("parallel",)),
    )(c, xt, w1t, b1t, w2t, b2t, zft, l1wst, l1wat, l1bt, l2wst, l2wat, l2bt)
    return out_t.T


def kernel(x, edge_index, z, params):
    src = edge_index[0]
    dst = edge_index[1]
    c = jnp.zeros((_NN, _NN), jnp.float32).at[dst, src].add(1.0)
    xt = x.T
    w1t = params["gauge_W1"].transpose(0, 2, 1).reshape(_NN * _HID, _IN)
    b1t = params["gauge_b1"].reshape(_NN * _HID, 1)
    w2t = params["gauge_W2"].transpose(0, 2, 1).reshape(_NN * _HID, _HID)
    b2t = params["gauge_b2"].T
    zft = (z @ params["z_W"] + params["z_b"]).reshape(_HID, 1)
    out_t = _run(
        c, xt, w1t, b1t, w2t, b2t, zft,
        params["l1_Ws"].T, params["l1_Wa"].T, params["l1_b"].reshape(_HID, 1),
        params["l2_Ws"].T, params["l2_Wa"].T, params["l2_b"].reshape(_OUT, 1),
    )
    return out_t
